# baseline (speedup 1.0000x reference)
import functools

import jax
import jax.numpy as jnp
from jax import lax
from jax.experimental import pallas as pl
from jax.experimental.pallas import tpu as pltpu
from jax.experimental.pallas import tpu_sc as plsc

EMBED = 64


@functools.partial(jax.jit, static_argnames=("total_rows",))
def _sc_gather(idx_flat, table, total_rows):
    info = plsc.get_sparse_core_info()
    nc, ns = info.num_cores, info.num_subcores
    nw = nc * ns
    b_per_w = total_rows // nw
    chunk = 1664
    n_chunks = b_per_w // chunk
    assert b_per_w % chunk == 0 and chunk % 8 == 0

    mesh = plsc.VectorSubcoreMesh(core_axis_name="c", subcore_axis_name="s")

    @functools.partial(
        pl.kernel,
        mesh=mesh,
        compiler_params=pltpu.CompilerParams(use_tc_tiling_on_sc=False),
        out_type=jax.ShapeDtypeStruct((total_rows, EMBED), jnp.float32),
        scratch_types=[
            pltpu.VMEM((chunk,), jnp.int32),
            pltpu.VMEM((chunk, EMBED), jnp.float32),
            pltpu.SemaphoreType.DMA,
        ],
    )
    def k(idx_hbm, table_hbm, out_hbm, idx_v, rows_v, sem):
        wid = lax.axis_index("s") * nc + lax.axis_index("c")
        base = wid * b_per_w

        def body(g, carry):
            off = base + g * chunk
            pltpu.sync_copy(idx_hbm.at[pl.ds(off, chunk)], idx_v)
            pltpu.async_copy(table_hbm.at[idx_v], rows_v, sem).wait()
            pltpu.sync_copy(rows_v, out_hbm.at[pl.ds(off, chunk)])
            return carry

        lax.fori_loop(0, n_chunks, body, 0)

    return k(idx_flat, table)


def kernel(indices, table):
    original_shape = indices.shape
    idx_flat = indices.reshape(-1)
    out = _sc_gather(idx_flat, table, idx_flat.shape[0])
    if len(original_shape) == 1:
        return out
    return out.reshape(*original_shape, table.shape[1])


# speedup vs baseline: 1.0027x; 1.0027x over previous
import functools

import jax
import jax.numpy as jnp
from jax import lax
from jax.experimental import pallas as pl
from jax.experimental.pallas import tpu as pltpu
from jax.experimental.pallas import tpu_sc as plsc

EMBED = 64
NBUF = 2


@functools.partial(jax.jit, static_argnames=("total_rows",))
def _sc_gather(idx_flat, table, total_rows):
    info = plsc.get_sparse_core_info()
    nc, ns = info.num_cores, info.num_subcores
    nw = nc * ns
    b_per_w = total_rows // nw
    chunk = 832
    n_chunks = b_per_w // chunk
    assert b_per_w % chunk == 0 and chunk % 8 == 0

    mesh = plsc.VectorSubcoreMesh(core_axis_name="c", subcore_axis_name="s")

    @functools.partial(
        pl.kernel,
        mesh=mesh,
        compiler_params=pltpu.CompilerParams(use_tc_tiling_on_sc=False),
        out_type=jax.ShapeDtypeStruct((total_rows, EMBED), jnp.float32),
        scratch_types=[
            pltpu.VMEM((b_per_w,), jnp.int32),
            pltpu.VMEM((NBUF, chunk, EMBED), jnp.float32),
            pltpu.SemaphoreType.DMA((NBUF,)),
            pltpu.SemaphoreType.DMA((NBUF,)),
        ],
    )
    def k(idx_hbm, table_hbm, out_hbm, idx_v, rows_v, gsem, ssem):
        wid = lax.axis_index("s") * nc + lax.axis_index("c")
        base = wid * b_per_w
        pltpu.sync_copy(idx_hbm.at[pl.ds(base, b_per_w)], idx_v)

        hg = [None] * NBUF
        hs = [None] * NBUF
        for g in range(n_chunks):
            b = g % NBUF
            if hs[b] is not None:
                hs[b].wait()
                hs[b] = None
            hg[b] = pltpu.async_copy(
                table_hbm.at[idx_v.at[pl.ds(g * chunk, chunk)]],
                rows_v.at[b],
                gsem.at[b],
            )
            d = g - (NBUF - 1)
            if d >= 0:
                bd = d % NBUF
                hg[bd].wait()
                hs[bd] = pltpu.async_copy(
                    rows_v.at[bd],
                    out_hbm.at[pl.ds(base + d * chunk, chunk)],
                    ssem.at[bd],
                )
        for d in range(max(0, n_chunks - (NBUF - 1)), n_chunks):
            bd = d % NBUF
            if hs[bd] is not None:
                hs[bd].wait()
                hs[bd] = None
            hg[bd].wait()
            hs[bd] = pltpu.async_copy(
                rows_v.at[bd],
                out_hbm.at[pl.ds(base + d * chunk, chunk)],
                ssem.at[bd],
            )
        for h in hs:
            if h is not None:
                h.wait()

    return k(idx_flat, table)


def kernel(indices, table):
    original_shape = indices.shape
    idx_flat = indices.reshape(-1)
    out = _sc_gather(idx_flat, table, idx_flat.shape[0])
    if len(original_shape) == 1:
        return out
    return out.reshape(*original_shape, table.shape[1])
